# Initial kernel scaffold; baseline (speedup 1.0000x reference)
import jax
import jax.numpy as jnp
from jax.experimental import pallas as pl


def kernel(x, edge_index, W_pre0, b_pre0, W_pre1, b_pre1, g_bn1, be_bn1, W_conv, b_conv, W_aft, b_aft, g_bn2, be_bn2, W_out, b_out):
    raise NotImplementedError("write your pallas kernel here")


# speedup vs baseline: 25.4308x; 25.4308x over previous
import functools

import jax
import jax.numpy as jnp
from jax import lax
from jax.experimental import pallas as pl
from jax.experimental.pallas import tpu as pltpu
from jax.experimental.pallas import tpu_sc as plsc

N = 50000
E = 800000
D_IN = 128
DH = 64
HF = 32
C_OUT = 40

R = 2000
GN = N // R
NS = 16
EPT = E // NS
EG = 400
NCH = EPT // EG
NP = 50048
RPT = NP // NS
DEG_EG = 1000
DEG_NCH = (EPT // 2) // DEG_EG

_f32 = jnp.float32


_sc_mesh = plsc.VectorSubcoreMesh(core_axis_name="c", subcore_axis_name="s")
_sc_params = pltpu.CompilerParams(use_tc_tiling_on_sc=False)


def _deg_body(dst_hbm, init_hbm, ones_hbm, deg_out, deg_sh, dst_v, ones_v):
    c = lax.axis_index("c")
    s = lax.axis_index("s")
    pltpu.sync_copy(ones_hbm, ones_v)
    pltpu.sync_copy(init_hbm.at[c], deg_sh.at[pl.ds(s * RPT, RPT)])
    plsc.subcore_barrier()
    base = c * (E // 2) + s * (EPT // 2)

    def body(k, carry):
        pltpu.sync_copy(dst_hbm.at[pl.ds(base + k * DEG_EG, DEG_EG)], dst_v)
        pltpu.sync_copy(ones_v, deg_sh.at[dst_v], add=True)
        return carry

    lax.fori_loop(0, DEG_NCH, body, 0)
    plsc.subcore_barrier()
    pltpu.sync_copy(deg_sh.at[pl.ds(s * RPT, RPT)],
                    deg_out.at[pl.ds(c * NP + s * RPT, RPT)])


_deg_call = pl.kernel(
    _deg_body,
    out_type=jax.ShapeDtypeStruct((2 * NP, 1), _f32),
    mesh=_sc_mesh,
    scratch_types=[
        pltpu.VMEM_SHARED((NP, 1), _f32),
        pltpu.VMEM((DEG_EG,), jnp.int32),
        pltpu.VMEM((DEG_EG, 1), _f32),
    ],
    compiler_params=_sc_params,
)


def _gat_body(idx_hbm, dst_hbm, tab_hbm, zr_hbm, s_out,
              s_sh, idx_v, dst_v, rows_v, sem):
    c = lax.axis_index("c")
    s = lax.axis_index("s")
    pltpu.sync_copy(zr_hbm, s_sh.at[pl.ds(s * RPT, RPT)])
    plsc.subcore_barrier()
    ibase = c * E + s * EPT
    dbase = s * EPT

    def body(k, carry):
        pltpu.sync_copy(idx_hbm.at[pl.ds(ibase + k * EG, EG)], idx_v)
        pltpu.sync_copy(dst_hbm.at[pl.ds(dbase + k * EG, EG)], dst_v)
        pltpu.async_copy(tab_hbm.at[idx_v], rows_v, sem).wait()
        pltpu.sync_copy(rows_v, s_sh.at[dst_v], add=True)
        return carry

    lax.fori_loop(0, NCH, body, 0)
    plsc.subcore_barrier()
    pltpu.sync_copy(s_sh.at[pl.ds(s * RPT, RPT)],
                    s_out.at[pl.ds(c * NP + s * RPT, RPT)])


_gat_call = pl.kernel(
    _gat_body,
    out_type=jax.ShapeDtypeStruct((2 * NP, HF), _f32),
    mesh=_sc_mesh,
    scratch_types=[
        pltpu.VMEM_SHARED((NP, HF), _f32),
        pltpu.VMEM((EG,), jnp.int32),
        pltpu.VMEM((EG,), jnp.int32),
        pltpu.VMEM((EG, HF), _f32),
        pltpu.SemaphoreType.DMA,
    ],
    compiler_params=_sc_params,
)


def _k1(x_ref, w0_ref, b0_ref, w1_ref, b1_ref, h2_ref, sm_ref, sq_ref):
    i = pl.program_id(0)
    h = jnp.dot(x_ref[...], w0_ref[...], preferred_element_type=_f32) + b0_ref[...]
    h = jnp.dot(h, w1_ref[...], preferred_element_type=_f32) + b1_ref[...]
    h = jnp.maximum(h, 0.0)
    h2_ref[...] = h

    @pl.when(i == 0)
    def _():
        sm_ref[...] = jnp.zeros_like(sm_ref)
        sq_ref[...] = jnp.zeros_like(sq_ref)

    sm_ref[...] += jnp.sum(h, axis=0, keepdims=True)
    sq_ref[...] += jnp.sum(h * h, axis=0, keepdims=True)


_full = lambda i: (0, 0)

_k1_call = pl.pallas_call(
    _k1,
    grid=(GN,),
    in_specs=[
        pl.BlockSpec((R, D_IN), lambda i: (i, 0)),
        pl.BlockSpec((D_IN, DH), _full),
        pl.BlockSpec((1, DH), _full),
        pl.BlockSpec((DH, DH), _full),
        pl.BlockSpec((1, DH), _full),
    ],
    out_specs=[
        pl.BlockSpec((R, DH), lambda i: (i, 0)),
        pl.BlockSpec((1, DH), _full),
        pl.BlockSpec((1, DH), _full),
    ],
    out_shape=[
        jax.ShapeDtypeStruct((N, DH), _f32),
        jax.ShapeDtypeStruct((1, DH), _f32),
        jax.ShapeDtypeStruct((1, DH), _f32),
    ],
)


def _bn_scale_shift(sm, sq, g, be):
    mu = sm / N
    var = sq / N - mu * mu
    sc = g * lax.rsqrt(var + 1e-5)
    return sc, be - mu * sc


def _k2(h2_ref, d0_ref, d1_ref, sm_ref, sq_ref, g_ref, be_ref, hp_ref):
    sc, sh = _bn_scale_shift(sm_ref[...], sq_ref[...], g_ref[...], be_ref[...])
    h = h2_ref[...] * sc + sh
    dinv = lax.rsqrt(d0_ref[...] + d1_ref[...])
    hp = h * dinv
    hp_ref[0] = hp[:, :HF]
    hp_ref[1] = hp[:, HF:]


_k2_call = pl.pallas_call(
    _k2,
    grid=(GN,),
    in_specs=[
        pl.BlockSpec((R, DH), lambda i: (i, 0)),
        pl.BlockSpec((R, 1), lambda i: (i, 0)),
        pl.BlockSpec((R, 1), lambda i: (i, 0)),
        pl.BlockSpec((1, DH), _full),
        pl.BlockSpec((1, DH), _full),
        pl.BlockSpec((1, DH), _full),
        pl.BlockSpec((1, DH), _full),
    ],
    out_specs=[pl.BlockSpec((2, R, HF), lambda i: (0, i, 0))],
    out_shape=[jax.ShapeDtypeStruct((2, N, HF), _f32)],
)


def _k3(h2_ref, s0_ref, s1_ref, d0_ref, d1_ref, sm_ref, sq_ref, g_ref, be_ref,
        wc_ref, bc_ref, wa_ref, ba_ref, h4_ref, sm2_ref, sq2_ref):
    i = pl.program_id(0)
    sc, sh = _bn_scale_shift(sm_ref[...], sq_ref[...], g_ref[...], be_ref[...])
    h = h2_ref[...] * sc + sh
    dinv = lax.rsqrt(d0_ref[...] + d1_ref[...])
    hp = h * dinv
    s_all = jnp.concatenate([s0_ref[...], s1_ref[...]], axis=1)
    agg = (s_all + hp) * dinv
    h3 = jnp.maximum(
        jnp.dot(agg, wc_ref[...], preferred_element_type=_f32) + bc_ref[...], 0.0)
    h4 = jnp.maximum(
        jnp.dot(h3, wa_ref[...], preferred_element_type=_f32) + ba_ref[...], 0.0)
    h4_ref[...] = h4

    @pl.when(i == 0)
    def _():
        sm2_ref[...] = jnp.zeros_like(sm2_ref)
        sq2_ref[...] = jnp.zeros_like(sq2_ref)

    sm2_ref[...] += jnp.sum(h4, axis=0, keepdims=True)
    sq2_ref[...] += jnp.sum(h4 * h4, axis=0, keepdims=True)


_k3_call = pl.pallas_call(
    _k3,
    grid=(GN,),
    in_specs=[
        pl.BlockSpec((R, DH), lambda i: (i, 0)),
        pl.BlockSpec((R, HF), lambda i: (i, 0)),
        pl.BlockSpec((R, HF), lambda i: (i, 0)),
        pl.BlockSpec((R, 1), lambda i: (i, 0)),
        pl.BlockSpec((R, 1), lambda i: (i, 0)),
        pl.BlockSpec((1, DH), _full),
        pl.BlockSpec((1, DH), _full),
        pl.BlockSpec((1, DH), _full),
        pl.BlockSpec((1, DH), _full),
        pl.BlockSpec((DH, DH), _full),
        pl.BlockSpec((1, DH), _full),
        pl.BlockSpec((DH, DH), _full),
        pl.BlockSpec((1, DH), _full),
    ],
    out_specs=[
        pl.BlockSpec((R, DH), lambda i: (i, 0)),
        pl.BlockSpec((1, DH), _full),
        pl.BlockSpec((1, DH), _full),
    ],
    out_shape=[
        jax.ShapeDtypeStruct((N, DH), _f32),
        jax.ShapeDtypeStruct((1, DH), _f32),
        jax.ShapeDtypeStruct((1, DH), _f32),
    ],
)


def _k4(h4_ref, sm_ref, sq_ref, g_ref, be_ref, wo_ref, bo_ref, o_ref):
    sc, sh = _bn_scale_shift(sm_ref[...], sq_ref[...], g_ref[...], be_ref[...])
    h = h4_ref[...] * sc + sh
    o_ref[...] = jnp.dot(h, wo_ref[...], preferred_element_type=_f32) + bo_ref[...]


_k4_call = pl.pallas_call(
    _k4,
    grid=(GN,),
    in_specs=[
        pl.BlockSpec((R, DH), lambda i: (i, 0)),
        pl.BlockSpec((1, DH), _full),
        pl.BlockSpec((1, DH), _full),
        pl.BlockSpec((1, DH), _full),
        pl.BlockSpec((1, DH), _full),
        pl.BlockSpec((DH, C_OUT), _full),
        pl.BlockSpec((1, C_OUT), _full),
    ],
    out_specs=[pl.BlockSpec((R, C_OUT), lambda i: (i, 0))],
    out_shape=[jax.ShapeDtypeStruct((N, C_OUT), _f32)],
)


def kernel(x, edge_index, W_pre0, b_pre0, W_pre1, b_pre1, g_bn1, be_bn1,
           W_conv, b_conv, W_aft, b_aft, g_bn2, be_bn2, W_out, b_out):
    src = edge_index[0]
    dst = edge_index[1]
    idx2 = jnp.concatenate([src, src + N])
    deg_init = jnp.concatenate(
        [jnp.ones((1, RPT, 1), _f32), jnp.zeros((1, RPT, 1), _f32)])
    ones_eg = jnp.ones((DEG_EG, 1), _f32)
    zrows = jnp.zeros((RPT, HF), _f32)

    deg2 = _deg_call(dst, deg_init, ones_eg)
    d0 = deg2[:N]
    d1 = deg2[NP:NP + N]
    h2, sm1, sq1 = _k1_call(
        x, W_pre0.T, b_pre0.reshape(1, DH), W_pre1.T, b_pre1.reshape(1, DH))
    g1 = g_bn1.reshape(1, DH)
    be1 = be_bn1.reshape(1, DH)
    (hp_tab,) = _k2_call(h2, d0, d1, sm1, sq1, g1, be1)
    s2 = _gat_call(idx2, dst, hp_tab.reshape(2 * N, HF), zrows)
    s0 = s2[:N]
    s1 = s2[NP:NP + N]
    h4, sm2, sq2 = _k3_call(
        h2, s0, s1, d0, d1, sm1, sq1, g1, be1,
        W_conv.T, b_conv.reshape(1, DH), W_aft.T, b_aft.reshape(1, DH))
    (out,) = _k4_call(
        h4, sm2, sq2, g_bn2.reshape(1, DH), be_bn2.reshape(1, DH),
        W_out.T, b_out.reshape(1, C_OUT))
    return out


# speedup vs baseline: 31.5384x; 1.2402x over previous
import jax
import jax.numpy as jnp
from jax import lax
from jax.experimental import pallas as pl
from jax.experimental.pallas import tpu as pltpu
from jax.experimental.pallas import tpu_sc as plsc

N = 50000
E = 800000
D_IN = 128
DH = 64
HF = 32
C_OUT = 40

R = 2000
GN = N // R
NS = 16
EPT = E // NS
EG = 400
PIPE = 2
NCH = EPT // EG
NPAIR = NCH // PIPE
NP = 50048
RPT = NP // NS
DEG_EG = 1000
DEG_NCH = (EPT // 2) // DEG_EG

_f32 = jnp.float32


_sc_mesh = plsc.VectorSubcoreMesh(core_axis_name="c", subcore_axis_name="s")
_sc_params = pltpu.CompilerParams(use_tc_tiling_on_sc=False)


def _deg_body(ei_hbm, init_hbm, ones_hbm, degA, degB, deg_sh, dst_v, ones_v):
    c = lax.axis_index("c")
    s = lax.axis_index("s")
    pltpu.sync_copy(ones_hbm, ones_v)
    pltpu.sync_copy(init_hbm.at[c], deg_sh.at[pl.ds(s * RPT, RPT)])
    plsc.subcore_barrier()
    base = c * (E // 2) + s * (EPT // 2)

    def body(k, carry):
        pltpu.sync_copy(ei_hbm.at[1, pl.ds(base + k * DEG_EG, DEG_EG)], dst_v)
        pltpu.sync_copy(ones_v, deg_sh.at[dst_v], add=True)
        return carry

    lax.fori_loop(0, DEG_NCH, body, 0)
    plsc.subcore_barrier()

    @pl.when(c == 0)
    def _():
        pltpu.sync_copy(deg_sh.at[pl.ds(s * RPT, RPT)],
                        degA.at[pl.ds(s * RPT, RPT)])

    @pl.when(c == 1)
    def _():
        pltpu.sync_copy(deg_sh.at[pl.ds(s * RPT, RPT)],
                        degB.at[pl.ds(s * RPT, RPT)])


_deg_call = pl.kernel(
    _deg_body,
    out_type=[jax.ShapeDtypeStruct((NP, 1), _f32),
              jax.ShapeDtypeStruct((NP, 1), _f32)],
    mesh=_sc_mesh,
    scratch_types=[
        pltpu.VMEM_SHARED((NP, 1), _f32),
        pltpu.VMEM((DEG_EG,), jnp.int32),
        pltpu.VMEM((DEG_EG, 1), _f32),
    ],
    compiler_params=_sc_params,
)


def _gat_body(ei_hbm, tab_hbm, zr_hbm, outA, outB, s_sh,
              i0, i1, d0, d1, r0, r1, m0, m1):
    c = lax.axis_index("c")
    s = lax.axis_index("s")
    pltpu.sync_copy(zr_hbm, s_sh.at[pl.ds(s * RPT, RPT)])
    plsc.subcore_barrier()
    ebase = s * EPT
    off16 = jnp.full((16,), c * N, jnp.int32)

    idx_bufs = (i0, i1)
    dst_bufs = (d0, d1)
    row_bufs = (r0, r1)
    sems = (m0, m1)

    def start_chunk(k, idx_v, dst_v, rows_v, sem):
        pltpu.sync_copy(ei_hbm.at[0, pl.ds(ebase + k * EG, EG)], idx_v)
        for jj in range(EG // 16):
            sl = pl.ds(jj * 16, 16)
            idx_v[sl] = idx_v[sl] + off16
        pltpu.sync_copy(ei_hbm.at[1, pl.ds(ebase + k * EG, EG)], dst_v)
        return pltpu.async_copy(tab_hbm.at[idx_v], rows_v, sem)

    def body(kp, carry):
        b = kp * PIPE
        descs = []
        for p in range(PIPE):
            descs.append(
                start_chunk(b + p, idx_bufs[p], dst_bufs[p], row_bufs[p],
                            sems[p]))
        for p in range(PIPE):
            descs[p].wait()
            pltpu.sync_copy(row_bufs[p], s_sh.at[dst_bufs[p]], add=True)
        return carry

    lax.fori_loop(0, NPAIR, body, 0)
    tail = start_chunk(NCH - 1, i0, d0, r0, m0)
    tail.wait()
    pltpu.sync_copy(r0, s_sh.at[d0], add=True)
    plsc.subcore_barrier()

    @pl.when(c == 0)
    def _():
        pltpu.sync_copy(s_sh.at[pl.ds(s * RPT, RPT)],
                        outA.at[pl.ds(s * RPT, RPT)])

    @pl.when(c == 1)
    def _():
        pltpu.sync_copy(s_sh.at[pl.ds(s * RPT, RPT)],
                        outB.at[pl.ds(s * RPT, RPT)])


_gat_call = pl.kernel(
    _gat_body,
    out_type=[jax.ShapeDtypeStruct((NP, HF), _f32),
              jax.ShapeDtypeStruct((NP, HF), _f32)],
    mesh=_sc_mesh,
    scratch_types=(
        [pltpu.VMEM_SHARED((NP, HF), _f32)]
        + [pltpu.VMEM((EG,), jnp.int32) for _ in range(PIPE)]
        + [pltpu.VMEM((EG,), jnp.int32) for _ in range(PIPE)]
        + [pltpu.VMEM((EG, HF), _f32) for _ in range(PIPE)]
        + [pltpu.SemaphoreType.DMA for _ in range(PIPE)]
    ),
    compiler_params=_sc_params,
)


def _k1(x_ref, w0_ref, b0_ref, w1_ref, b1_ref, h2_ref, sm_ref, sq_ref):
    i = pl.program_id(0)
    h = jnp.dot(x_ref[...], w0_ref[...], preferred_element_type=_f32) + b0_ref[...]
    h = jnp.dot(h, w1_ref[...], preferred_element_type=_f32) + b1_ref[...]
    h = jnp.maximum(h, 0.0)
    h2_ref[...] = h

    @pl.when(i == 0)
    def _():
        sm_ref[...] = jnp.zeros_like(sm_ref)
        sq_ref[...] = jnp.zeros_like(sq_ref)

    sm_ref[...] += jnp.sum(h, axis=0, keepdims=True)
    sq_ref[...] += jnp.sum(h * h, axis=0, keepdims=True)


_full = lambda *_: (0, 0)

_k1_call = pl.pallas_call(
    _k1,
    grid=(GN,),
    in_specs=[
        pl.BlockSpec((R, D_IN), lambda i: (i, 0)),
        pl.BlockSpec((D_IN, DH), _full),
        pl.BlockSpec((1, DH), _full),
        pl.BlockSpec((DH, DH), _full),
        pl.BlockSpec((1, DH), _full),
    ],
    out_specs=[
        pl.BlockSpec((R, DH), lambda i: (i, 0)),
        pl.BlockSpec((1, DH), _full),
        pl.BlockSpec((1, DH), _full),
    ],
    out_shape=[
        jax.ShapeDtypeStruct((N, DH), _f32),
        jax.ShapeDtypeStruct((1, DH), _f32),
        jax.ShapeDtypeStruct((1, DH), _f32),
    ],
)


def _bn_scale_shift(sm, sq, g, be):
    mu = sm / N
    var = sq / N - mu * mu
    sc = g * lax.rsqrt(var + 1e-5)
    return sc, be - mu * sc


def _k2(h2_ref, d0_ref, d1_ref, sm_ref, sq_ref, g_ref, be_ref, hp_ref):
    j = pl.program_id(0)
    sc, sh = _bn_scale_shift(sm_ref[...], sq_ref[...], g_ref[...], be_ref[...])
    h = h2_ref[...] * sc + sh
    dinv = lax.rsqrt(d0_ref[...] + d1_ref[...])
    hp = h * dinv
    hp_ref[...] = jnp.where(j == 0, hp[:, :HF], hp[:, HF:])


_k2_call = pl.pallas_call(
    _k2,
    grid=(2, GN),
    in_specs=[
        pl.BlockSpec((R, DH), lambda j, i: (i, 0)),
        pl.BlockSpec((R, 1), lambda j, i: (i, 0)),
        pl.BlockSpec((R, 1), lambda j, i: (i, 0)),
        pl.BlockSpec((1, DH), _full),
        pl.BlockSpec((1, DH), _full),
        pl.BlockSpec((1, DH), _full),
        pl.BlockSpec((1, DH), _full),
    ],
    out_specs=[pl.BlockSpec((R, HF), lambda j, i: (j * GN + i, 0))],
    out_shape=[jax.ShapeDtypeStruct((2 * N, HF), _f32)],
)


def _k3(h2_ref, s0_ref, s1_ref, d0_ref, d1_ref, sm_ref, sq_ref, g_ref, be_ref,
        wc_ref, bc_ref, wa_ref, ba_ref, h4_ref, sm2_ref, sq2_ref):
    i = pl.program_id(0)
    sc, sh = _bn_scale_shift(sm_ref[...], sq_ref[...], g_ref[...], be_ref[...])
    h = h2_ref[...] * sc + sh
    dinv = lax.rsqrt(d0_ref[...] + d1_ref[...])
    hp = h * dinv
    s_all = jnp.concatenate([s0_ref[...], s1_ref[...]], axis=1)
    agg = (s_all + hp) * dinv
    h3 = jnp.maximum(
        jnp.dot(agg, wc_ref[...], preferred_element_type=_f32) + bc_ref[...], 0.0)
    h4 = jnp.maximum(
        jnp.dot(h3, wa_ref[...], preferred_element_type=_f32) + ba_ref[...], 0.0)
    h4_ref[...] = h4

    @pl.when(i == 0)
    def _():
        sm2_ref[...] = jnp.zeros_like(sm2_ref)
        sq2_ref[...] = jnp.zeros_like(sq2_ref)

    sm2_ref[...] += jnp.sum(h4, axis=0, keepdims=True)
    sq2_ref[...] += jnp.sum(h4 * h4, axis=0, keepdims=True)


_k3_call = pl.pallas_call(
    _k3,
    grid=(GN,),
    in_specs=[
        pl.BlockSpec((R, DH), lambda i: (i, 0)),
        pl.BlockSpec((R, HF), lambda i: (i, 0)),
        pl.BlockSpec((R, HF), lambda i: (i, 0)),
        pl.BlockSpec((R, 1), lambda i: (i, 0)),
        pl.BlockSpec((R, 1), lambda i: (i, 0)),
        pl.BlockSpec((1, DH), _full),
        pl.BlockSpec((1, DH), _full),
        pl.BlockSpec((1, DH), _full),
        pl.BlockSpec((1, DH), _full),
        pl.BlockSpec((DH, DH), _full),
        pl.BlockSpec((1, DH), _full),
        pl.BlockSpec((DH, DH), _full),
        pl.BlockSpec((1, DH), _full),
    ],
    out_specs=[
        pl.BlockSpec((R, DH), lambda i: (i, 0)),
        pl.BlockSpec((1, DH), _full),
        pl.BlockSpec((1, DH), _full),
    ],
    out_shape=[
        jax.ShapeDtypeStruct((N, DH), _f32),
        jax.ShapeDtypeStruct((1, DH), _f32),
        jax.ShapeDtypeStruct((1, DH), _f32),
    ],
)


def _k4(h4_ref, sm_ref, sq_ref, g_ref, be_ref, wo_ref, bo_ref, o_ref):
    sc, sh = _bn_scale_shift(sm_ref[...], sq_ref[...], g_ref[...], be_ref[...])
    h = h4_ref[...] * sc + sh
    o_ref[...] = jnp.dot(h, wo_ref[...], preferred_element_type=_f32) + bo_ref[...]


_k4_call = pl.pallas_call(
    _k4,
    grid=(GN,),
    in_specs=[
        pl.BlockSpec((R, DH), lambda i: (i, 0)),
        pl.BlockSpec((1, DH), _full),
        pl.BlockSpec((1, DH), _full),
        pl.BlockSpec((1, DH), _full),
        pl.BlockSpec((1, DH), _full),
        pl.BlockSpec((DH, C_OUT), _full),
        pl.BlockSpec((1, C_OUT), _full),
    ],
    out_specs=[pl.BlockSpec((R, C_OUT), lambda i: (i, 0))],
    out_shape=[jax.ShapeDtypeStruct((N, C_OUT), _f32)],
)


def kernel(x, edge_index, W_pre0, b_pre0, W_pre1, b_pre1, g_bn1, be_bn1,
           W_conv, b_conv, W_aft, b_aft, g_bn2, be_bn2, W_out, b_out):
    deg_init = jnp.concatenate(
        [jnp.ones((1, RPT, 1), _f32), jnp.zeros((1, RPT, 1), _f32)])
    ones_eg = jnp.ones((DEG_EG, 1), _f32)
    zrows = jnp.zeros((RPT, HF), _f32)

    degA, degB = _deg_call(edge_index, deg_init, ones_eg)
    h2, sm1, sq1 = _k1_call(
        x, W_pre0.T, b_pre0.reshape(1, DH), W_pre1.T, b_pre1.reshape(1, DH))
    g1 = g_bn1.reshape(1, DH)
    be1 = be_bn1.reshape(1, DH)
    (hp_tab,) = _k2_call(h2, degA, degB, sm1, sq1, g1, be1)
    sA, sB = _gat_call(edge_index, hp_tab, zrows)
    h4, sm2, sq2 = _k3_call(
        h2, sA, sB, degA, degB, sm1, sq1, g1, be1,
        W_conv.T, b_conv.reshape(1, DH), W_aft.T, b_aft.reshape(1, DH))
    (out,) = _k4_call(
        h4, sm2, sq2, g_bn2.reshape(1, DH), be_bn2.reshape(1, DH),
        W_out.T, b_out.reshape(1, C_OUT))
    return out


# speedup vs baseline: 35.1702x; 1.1152x over previous
import jax
import jax.numpy as jnp
from jax import lax
from jax.experimental import pallas as pl
from jax.experimental.pallas import tpu as pltpu
from jax.experimental.pallas import tpu_sc as plsc

N = 50000
E = 800000
D_IN = 128
DH = 64
HF = 32
C_OUT = 40

R = 5000
GN = N // R
NS = 16
EPT = E // NS
EG = 400
PIPE = 2
NCH = EPT // EG
NPAIR = NCH // PIPE
NP = 50048
RPT = NP // NS
DEG_EG = 1000
DEG_NCH = (EPT // 2) // DEG_EG

_f32 = jnp.float32


_sc_mesh = plsc.VectorSubcoreMesh(core_axis_name="c", subcore_axis_name="s")
_sc_params = pltpu.CompilerParams(use_tc_tiling_on_sc=False)


def _deg_body(ei_hbm, init_hbm, ones_hbm, degA, degB, deg_sh, dst_v, ones_v):
    c = lax.axis_index("c")
    s = lax.axis_index("s")
    pltpu.sync_copy(ones_hbm, ones_v)
    pltpu.sync_copy(init_hbm.at[c], deg_sh.at[pl.ds(s * RPT, RPT)])
    plsc.subcore_barrier()
    base = c * (E // 2) + s * (EPT // 2)

    def body(k, carry):
        pltpu.sync_copy(ei_hbm.at[1, pl.ds(base + k * DEG_EG, DEG_EG)], dst_v)
        pltpu.sync_copy(ones_v, deg_sh.at[dst_v], add=True)
        return carry

    lax.fori_loop(0, DEG_NCH, body, 0)
    plsc.subcore_barrier()

    @pl.when(c == 0)
    def _():
        pltpu.sync_copy(deg_sh.at[pl.ds(s * RPT, RPT)],
                        degA.at[pl.ds(s * RPT, RPT)])

    @pl.when(c == 1)
    def _():
        pltpu.sync_copy(deg_sh.at[pl.ds(s * RPT, RPT)],
                        degB.at[pl.ds(s * RPT, RPT)])


_deg_call = pl.kernel(
    _deg_body,
    out_type=[jax.ShapeDtypeStruct((NP, 1), _f32),
              jax.ShapeDtypeStruct((NP, 1), _f32)],
    mesh=_sc_mesh,
    scratch_types=[
        pltpu.VMEM_SHARED((NP, 1), _f32),
        pltpu.VMEM((DEG_EG,), jnp.int32),
        pltpu.VMEM((DEG_EG, 1), _f32),
    ],
    compiler_params=_sc_params,
)


def _gat_body(ei_hbm, tab_hbm, zr_hbm, outA, outB, s_sh, *bufs):
    c = lax.axis_index("c")
    s = lax.axis_index("s")
    pltpu.sync_copy(zr_hbm, s_sh.at[pl.ds(s * RPT, RPT)])
    plsc.subcore_barrier()
    ebase = s * EPT
    off16 = jnp.full((16,), c * N, jnp.int32)

    idx_bufs = bufs[0:PIPE]
    dst_bufs = bufs[PIPE:2 * PIPE]
    row_bufs = bufs[2 * PIPE:3 * PIPE]
    sems = bufs[3 * PIPE:4 * PIPE]

    def start_chunk(k, idx_v, dst_v, rows_v, sem):
        pltpu.sync_copy(ei_hbm.at[0, pl.ds(ebase + k * EG, EG)], idx_v)
        for jj in range(EG // 16):
            sl = pl.ds(jj * 16, 16)
            idx_v[sl] = idx_v[sl] + off16
        pltpu.sync_copy(ei_hbm.at[1, pl.ds(ebase + k * EG, EG)], dst_v)
        pltpu.async_copy(tab_hbm.at[idx_v], rows_v, sem)

    def drain_scatter(p):
        pltpu.make_async_copy(
            tab_hbm.at[pl.ds(0, EG)], row_bufs[p], sems[p]).wait()
        pltpu.sync_copy(row_bufs[p], s_sh.at[dst_bufs[p]], add=True)

    for p in range(PIPE):
        start_chunk(p, idx_bufs[p], dst_bufs[p], row_bufs[p], sems[p])

    MAIN = (NCH - PIPE) // PIPE

    def body(q, carry):
        b = q * PIPE
        for p in range(PIPE):
            drain_scatter(p)
            k2 = b + p + PIPE
            start_chunk(k2, idx_bufs[p], dst_bufs[p], row_bufs[p], sems[p])
        return carry

    lax.fori_loop(0, MAIN, body, 0)
    for kk in range(MAIN * PIPE, NCH):
        p = kk % PIPE
        drain_scatter(p)
        if kk + PIPE < NCH:
            start_chunk(kk + PIPE, idx_bufs[p], dst_bufs[p], row_bufs[p],
                        sems[p])
    plsc.subcore_barrier()

    @pl.when(c == 0)
    def _():
        pltpu.sync_copy(s_sh.at[pl.ds(s * RPT, RPT)],
                        outA.at[pl.ds(s * RPT, RPT)])

    @pl.when(c == 1)
    def _():
        pltpu.sync_copy(s_sh.at[pl.ds(s * RPT, RPT)],
                        outB.at[pl.ds(s * RPT, RPT)])


_gat_call = pl.kernel(
    _gat_body,
    out_type=[jax.ShapeDtypeStruct((NP, HF), _f32),
              jax.ShapeDtypeStruct((NP, HF), _f32)],
    mesh=_sc_mesh,
    scratch_types=(
        [pltpu.VMEM_SHARED((NP, HF), _f32)]
        + [pltpu.VMEM((EG,), jnp.int32) for _ in range(PIPE)]
        + [pltpu.VMEM((EG,), jnp.int32) for _ in range(PIPE)]
        + [pltpu.VMEM((EG, HF), _f32) for _ in range(PIPE)]
        + [pltpu.SemaphoreType.DMA for _ in range(PIPE)]
    ),
    compiler_params=_sc_params,
)


def _k1(x_ref, w0_ref, b0_ref, w1_ref, b1_ref, h2_ref, sm_ref, sq_ref):
    i = pl.program_id(0)
    h = jnp.dot(x_ref[...], w0_ref[...], preferred_element_type=_f32) + b0_ref[...]
    h = jnp.dot(h, w1_ref[...], preferred_element_type=_f32) + b1_ref[...]
    h = jnp.maximum(h, 0.0)
    h2_ref[...] = h

    @pl.when(i == 0)
    def _():
        sm_ref[...] = jnp.zeros_like(sm_ref)
        sq_ref[...] = jnp.zeros_like(sq_ref)

    sm_ref[...] += jnp.sum(h, axis=0, keepdims=True)
    sq_ref[...] += jnp.sum(h * h, axis=0, keepdims=True)


_full = lambda *_: (0, 0)

_k1_call = pl.pallas_call(
    _k1,
    grid=(GN,),
    in_specs=[
        pl.BlockSpec((R, D_IN), lambda i: (i, 0)),
        pl.BlockSpec((D_IN, DH), _full),
        pl.BlockSpec((1, DH), _full),
        pl.BlockSpec((DH, DH), _full),
        pl.BlockSpec((1, DH), _full),
    ],
    out_specs=[
        pl.BlockSpec((R, DH), lambda i: (i, 0)),
        pl.BlockSpec((1, DH), _full),
        pl.BlockSpec((1, DH), _full),
    ],
    out_shape=[
        jax.ShapeDtypeStruct((N, DH), _f32),
        jax.ShapeDtypeStruct((1, DH), _f32),
        jax.ShapeDtypeStruct((1, DH), _f32),
    ],
)


def _bn_scale_shift(sm, sq, g, be):
    mu = sm / N
    var = sq / N - mu * mu
    sc = g * lax.rsqrt(var + 1e-5)
    return sc, be - mu * sc


def _k2(h2_ref, d0_ref, d1_ref, sm_ref, sq_ref, g_ref, be_ref, hp_ref):
    j = pl.program_id(0)
    sc, sh = _bn_scale_shift(sm_ref[...], sq_ref[...], g_ref[...], be_ref[...])
    h = h2_ref[...] * sc + sh
    dinv = lax.rsqrt(d0_ref[...] + d1_ref[...])
    hp = h * dinv
    hp_ref[...] = jnp.where(j == 0, hp[:, :HF], hp[:, HF:])


_k2_call = pl.pallas_call(
    _k2,
    grid=(2, GN),
    in_specs=[
        pl.BlockSpec((R, DH), lambda j, i: (i, 0)),
        pl.BlockSpec((R, 1), lambda j, i: (i, 0)),
        pl.BlockSpec((R, 1), lambda j, i: (i, 0)),
        pl.BlockSpec((1, DH), _full),
        pl.BlockSpec((1, DH), _full),
        pl.BlockSpec((1, DH), _full),
        pl.BlockSpec((1, DH), _full),
    ],
    out_specs=[pl.BlockSpec((R, HF), lambda j, i: (j * GN + i, 0))],
    out_shape=[jax.ShapeDtypeStruct((2 * N, HF), _f32)],
)


def _k3(h2_ref, s0_ref, s1_ref, d0_ref, d1_ref, sm_ref, sq_ref, g_ref, be_ref,
        wc_ref, bc_ref, wa_ref, ba_ref, h4_ref, sm2_ref, sq2_ref):
    i = pl.program_id(0)
    sc, sh = _bn_scale_shift(sm_ref[...], sq_ref[...], g_ref[...], be_ref[...])
    h = h2_ref[...] * sc + sh
    dinv = lax.rsqrt(d0_ref[...] + d1_ref[...])
    hp = h * dinv
    s_all = jnp.concatenate([s0_ref[...], s1_ref[...]], axis=1)
    agg = (s_all + hp) * dinv
    h3 = jnp.maximum(
        jnp.dot(agg, wc_ref[...], preferred_element_type=_f32) + bc_ref[...], 0.0)
    h4 = jnp.maximum(
        jnp.dot(h3, wa_ref[...], preferred_element_type=_f32) + ba_ref[...], 0.0)
    h4_ref[...] = h4

    @pl.when(i == 0)
    def _():
        sm2_ref[...] = jnp.zeros_like(sm2_ref)
        sq2_ref[...] = jnp.zeros_like(sq2_ref)

    sm2_ref[...] += jnp.sum(h4, axis=0, keepdims=True)
    sq2_ref[...] += jnp.sum(h4 * h4, axis=0, keepdims=True)


_k3_call = pl.pallas_call(
    _k3,
    grid=(GN,),
    in_specs=[
        pl.BlockSpec((R, DH), lambda i: (i, 0)),
        pl.BlockSpec((R, HF), lambda i: (i, 0)),
        pl.BlockSpec((R, HF), lambda i: (i, 0)),
        pl.BlockSpec((R, 1), lambda i: (i, 0)),
        pl.BlockSpec((R, 1), lambda i: (i, 0)),
        pl.BlockSpec((1, DH), _full),
        pl.BlockSpec((1, DH), _full),
        pl.BlockSpec((1, DH), _full),
        pl.BlockSpec((1, DH), _full),
        pl.BlockSpec((DH, DH), _full),
        pl.BlockSpec((1, DH), _full),
        pl.BlockSpec((DH, DH), _full),
        pl.BlockSpec((1, DH), _full),
    ],
    out_specs=[
        pl.BlockSpec((R, DH), lambda i: (i, 0)),
        pl.BlockSpec((1, DH), _full),
        pl.BlockSpec((1, DH), _full),
    ],
    out_shape=[
        jax.ShapeDtypeStruct((N, DH), _f32),
        jax.ShapeDtypeStruct((1, DH), _f32),
        jax.ShapeDtypeStruct((1, DH), _f32),
    ],
)


def _k4(h4_ref, sm_ref, sq_ref, g_ref, be_ref, wo_ref, bo_ref, o_ref):
    sc, sh = _bn_scale_shift(sm_ref[...], sq_ref[...], g_ref[...], be_ref[...])
    h = h4_ref[...] * sc + sh
    o_ref[...] = jnp.dot(h, wo_ref[...], preferred_element_type=_f32) + bo_ref[...]


_k4_call = pl.pallas_call(
    _k4,
    grid=(GN,),
    in_specs=[
        pl.BlockSpec((R, DH), lambda i: (i, 0)),
        pl.BlockSpec((1, DH), _full),
        pl.BlockSpec((1, DH), _full),
        pl.BlockSpec((1, DH), _full),
        pl.BlockSpec((1, DH), _full),
        pl.BlockSpec((DH, C_OUT), _full),
        pl.BlockSpec((1, C_OUT), _full),
    ],
    out_specs=[pl.BlockSpec((R, C_OUT), lambda i: (i, 0))],
    out_shape=[jax.ShapeDtypeStruct((N, C_OUT), _f32)],
)


def kernel(x, edge_index, W_pre0, b_pre0, W_pre1, b_pre1, g_bn1, be_bn1,
           W_conv, b_conv, W_aft, b_aft, g_bn2, be_bn2, W_out, b_out):
    deg_init = jnp.concatenate(
        [jnp.ones((1, RPT, 1), _f32), jnp.zeros((1, RPT, 1), _f32)])
    ones_eg = jnp.ones((DEG_EG, 1), _f32)
    zrows = jnp.zeros((RPT, HF), _f32)

    degA, degB = _deg_call(edge_index, deg_init, ones_eg)
    h2, sm1, sq1 = _k1_call(
        x, W_pre0.T, b_pre0.reshape(1, DH), W_pre1.T, b_pre1.reshape(1, DH))
    g1 = g_bn1.reshape(1, DH)
    be1 = be_bn1.reshape(1, DH)
    (hp_tab,) = _k2_call(h2, degA, degB, sm1, sq1, g1, be1)
    sA, sB = _gat_call(edge_index, hp_tab, zrows)
    h4, sm2, sq2 = _k3_call(
        h2, sA, sB, degA, degB, sm1, sq1, g1, be1,
        W_conv.T, b_conv.reshape(1, DH), W_aft.T, b_aft.reshape(1, DH))
    (out,) = _k4_call(
        h4, sm2, sq2, g_bn2.reshape(1, DH), be_bn2.reshape(1, DH),
        W_out.T, b_out.reshape(1, C_OUT))
    return out


# speedup vs baseline: 36.9077x; 1.0494x over previous
import jax
import jax.numpy as jnp
from jax import lax
from jax.experimental import pallas as pl
from jax.experimental.pallas import tpu as pltpu
from jax.experimental.pallas import tpu_sc as plsc

N = 50000
E = 800000
D_IN = 128
DH = 64
HF = 32
C_OUT = 40

R = 5000
GN = N // R
NS = 16
EPT = E // NS
EG = 400
PIPE = 2
NCH = EPT // EG
NPAIR = NCH // PIPE
NP = 50048
RPT = NP // NS
DEG_EG = 1000
DEG_NCH = (EPT // 2) // DEG_EG

_f32 = jnp.float32


_sc_mesh = plsc.VectorSubcoreMesh(core_axis_name="c", subcore_axis_name="s")
_sc_params = pltpu.CompilerParams(use_tc_tiling_on_sc=False)


def _deg_body(ei_hbm, init_hbm, ones_hbm, degA, degB, deg_sh, dst_v, ones_v):
    c = lax.axis_index("c")
    s = lax.axis_index("s")
    pltpu.sync_copy(ones_hbm, ones_v)
    pltpu.sync_copy(init_hbm.at[c], deg_sh.at[pl.ds(s * RPT, RPT)])
    plsc.subcore_barrier()
    base = c * (E // 2) + s * (EPT // 2)

    def body(k, carry):
        pltpu.sync_copy(ei_hbm.at[1, pl.ds(base + k * DEG_EG, DEG_EG)], dst_v)
        pltpu.sync_copy(ones_v, deg_sh.at[dst_v], add=True)
        return carry

    lax.fori_loop(0, DEG_NCH, body, 0)
    plsc.subcore_barrier()

    @pl.when(c == 0)
    def _():
        pltpu.sync_copy(deg_sh.at[pl.ds(s * RPT, RPT)],
                        degA.at[pl.ds(s * RPT, RPT)])

    @pl.when(c == 1)
    def _():
        pltpu.sync_copy(deg_sh.at[pl.ds(s * RPT, RPT)],
                        degB.at[pl.ds(s * RPT, RPT)])


_deg_call = pl.kernel(
    _deg_body,
    out_type=[jax.ShapeDtypeStruct((NP, 1), _f32),
              jax.ShapeDtypeStruct((NP, 1), _f32)],
    mesh=_sc_mesh,
    scratch_types=[
        pltpu.VMEM_SHARED((NP, 1), _f32),
        pltpu.VMEM((DEG_EG,), jnp.int32),
        pltpu.VMEM((DEG_EG, 1), _f32),
    ],
    compiler_params=_sc_params,
)


def _gat_body(ei_hbm, tab_hbm, zr_hbm, outA, outB, s_sh, *bufs):
    c = lax.axis_index("c")
    s = lax.axis_index("s")
    pltpu.sync_copy(zr_hbm, s_sh.at[pl.ds(s * RPT, RPT)])
    plsc.subcore_barrier()
    ebase = s * EPT

    idx_bufs = bufs[0:PIPE]
    dst_bufs = bufs[PIPE:2 * PIPE]
    row_bufs = bufs[2 * PIPE:3 * PIPE]
    sems = bufs[3 * PIPE:4 * PIPE]

    def start_chunk(k, idx_v, dst_v, rows_v, sem):
        pltpu.sync_copy(ei_hbm.at[0, pl.ds(ebase + k * EG, EG)], idx_v)
        pltpu.sync_copy(ei_hbm.at[1, pl.ds(ebase + k * EG, EG)], dst_v)
        pltpu.async_copy(tab_hbm.at[c].at[idx_v], rows_v, sem)

    def drain_scatter(p):
        pltpu.make_async_copy(
            tab_hbm.at[0, pl.ds(0, EG)], row_bufs[p], sems[p]).wait()
        pltpu.sync_copy(row_bufs[p], s_sh.at[dst_bufs[p]], add=True)

    for p in range(PIPE):
        start_chunk(p, idx_bufs[p], dst_bufs[p], row_bufs[p], sems[p])

    MAIN = (NCH - PIPE) // PIPE

    def body(q, carry):
        b = q * PIPE
        for p in range(PIPE):
            drain_scatter(p)
            k2 = b + p + PIPE
            start_chunk(k2, idx_bufs[p], dst_bufs[p], row_bufs[p], sems[p])
        return carry

    lax.fori_loop(0, MAIN, body, 0)
    for kk in range(MAIN * PIPE, NCH):
        p = kk % PIPE
        drain_scatter(p)
        if kk + PIPE < NCH:
            start_chunk(kk + PIPE, idx_bufs[p], dst_bufs[p], row_bufs[p],
                        sems[p])
    plsc.subcore_barrier()

    @pl.when(c == 0)
    def _():
        pltpu.sync_copy(s_sh.at[pl.ds(s * RPT, RPT)],
                        outA.at[pl.ds(s * RPT, RPT)])

    @pl.when(c == 1)
    def _():
        pltpu.sync_copy(s_sh.at[pl.ds(s * RPT, RPT)],
                        outB.at[pl.ds(s * RPT, RPT)])


_gat_call = pl.kernel(
    _gat_body,
    out_type=[jax.ShapeDtypeStruct((NP, HF), _f32),
              jax.ShapeDtypeStruct((NP, HF), _f32)],
    mesh=_sc_mesh,
    scratch_types=(
        [pltpu.VMEM_SHARED((NP, HF), _f32)]
        + [pltpu.VMEM((EG,), jnp.int32) for _ in range(PIPE)]
        + [pltpu.VMEM((EG,), jnp.int32) for _ in range(PIPE)]
        + [pltpu.VMEM((EG, HF), _f32) for _ in range(PIPE)]
        + [pltpu.SemaphoreType.DMA for _ in range(PIPE)]
    ),
    compiler_params=_sc_params,
)


def _k1(x_ref, w0_ref, b0_ref, w1_ref, b1_ref, h2_ref, sm_ref, sq_ref):
    i = pl.program_id(0)
    h = jnp.dot(x_ref[...], w0_ref[...], preferred_element_type=_f32) + b0_ref[...]
    h = jnp.dot(h, w1_ref[...], preferred_element_type=_f32) + b1_ref[...]
    h = jnp.maximum(h, 0.0)
    h2_ref[...] = h

    @pl.when(i == 0)
    def _():
        sm_ref[...] = jnp.zeros_like(sm_ref)
        sq_ref[...] = jnp.zeros_like(sq_ref)

    sm_ref[...] += jnp.sum(h, axis=0, keepdims=True)
    sq_ref[...] += jnp.sum(h * h, axis=0, keepdims=True)


_full = lambda *_: (0, 0)

_k1_call = pl.pallas_call(
    _k1,
    grid=(GN,),
    in_specs=[
        pl.BlockSpec((R, D_IN), lambda i: (i, 0)),
        pl.BlockSpec((D_IN, DH), _full),
        pl.BlockSpec((1, DH), _full),
        pl.BlockSpec((DH, DH), _full),
        pl.BlockSpec((1, DH), _full),
    ],
    out_specs=[
        pl.BlockSpec((R, DH), lambda i: (i, 0)),
        pl.BlockSpec((1, DH), _full),
        pl.BlockSpec((1, DH), _full),
    ],
    out_shape=[
        jax.ShapeDtypeStruct((N, DH), _f32),
        jax.ShapeDtypeStruct((1, DH), _f32),
        jax.ShapeDtypeStruct((1, DH), _f32),
    ],
)


def _bn_scale_shift(sm, sq, g, be):
    mu = sm / N
    var = sq / N - mu * mu
    sc = g * lax.rsqrt(var + 1e-5)
    return sc, be - mu * sc


def _k2(h2_ref, d0_ref, d1_ref, sm_ref, sq_ref, g_ref, be_ref, hp_ref):
    sc, sh = _bn_scale_shift(sm_ref[...], sq_ref[...], g_ref[...], be_ref[...])
    h = h2_ref[...] * sc + sh
    dinv = lax.rsqrt(d0_ref[...] + d1_ref[...])
    hp = h * dinv
    hp_ref[0] = hp[:, :HF]
    hp_ref[1] = hp[:, HF:]


_k2_call = pl.pallas_call(
    _k2,
    grid=(GN,),
    in_specs=[
        pl.BlockSpec((R, DH), lambda i: (i, 0)),
        pl.BlockSpec((R, 1), lambda i: (i, 0)),
        pl.BlockSpec((R, 1), lambda i: (i, 0)),
        pl.BlockSpec((1, DH), _full),
        pl.BlockSpec((1, DH), _full),
        pl.BlockSpec((1, DH), _full),
        pl.BlockSpec((1, DH), _full),
    ],
    out_specs=[pl.BlockSpec((2, R, HF), lambda i: (0, i, 0))],
    out_shape=[jax.ShapeDtypeStruct((2, N, HF), _f32)],
)


def _k3(h2_ref, s0_ref, s1_ref, d0_ref, d1_ref, sm_ref, sq_ref, g_ref, be_ref,
        wc_ref, bc_ref, wa_ref, ba_ref, h4_ref, sm2_ref, sq2_ref):
    i = pl.program_id(0)
    sc, sh = _bn_scale_shift(sm_ref[...], sq_ref[...], g_ref[...], be_ref[...])
    h = h2_ref[...] * sc + sh
    dinv = lax.rsqrt(d0_ref[...] + d1_ref[...])
    hp = h * dinv
    s_all = jnp.concatenate([s0_ref[...], s1_ref[...]], axis=1)
    agg = (s_all + hp) * dinv
    h3 = jnp.maximum(
        jnp.dot(agg, wc_ref[...], preferred_element_type=_f32) + bc_ref[...], 0.0)
    h4 = jnp.maximum(
        jnp.dot(h3, wa_ref[...], preferred_element_type=_f32) + ba_ref[...], 0.0)
    h4_ref[...] = h4

    @pl.when(i == 0)
    def _():
        sm2_ref[...] = jnp.zeros_like(sm2_ref)
        sq2_ref[...] = jnp.zeros_like(sq2_ref)

    sm2_ref[...] += jnp.sum(h4, axis=0, keepdims=True)
    sq2_ref[...] += jnp.sum(h4 * h4, axis=0, keepdims=True)


_k3_call = pl.pallas_call(
    _k3,
    grid=(GN,),
    in_specs=[
        pl.BlockSpec((R, DH), lambda i: (i, 0)),
        pl.BlockSpec((R, HF), lambda i: (i, 0)),
        pl.BlockSpec((R, HF), lambda i: (i, 0)),
        pl.BlockSpec((R, 1), lambda i: (i, 0)),
        pl.BlockSpec((R, 1), lambda i: (i, 0)),
        pl.BlockSpec((1, DH), _full),
        pl.BlockSpec((1, DH), _full),
        pl.BlockSpec((1, DH), _full),
        pl.BlockSpec((1, DH), _full),
        pl.BlockSpec((DH, DH), _full),
        pl.BlockSpec((1, DH), _full),
        pl.BlockSpec((DH, DH), _full),
        pl.BlockSpec((1, DH), _full),
    ],
    out_specs=[
        pl.BlockSpec((R, DH), lambda i: (i, 0)),
        pl.BlockSpec((1, DH), _full),
        pl.BlockSpec((1, DH), _full),
    ],
    out_shape=[
        jax.ShapeDtypeStruct((N, DH), _f32),
        jax.ShapeDtypeStruct((1, DH), _f32),
        jax.ShapeDtypeStruct((1, DH), _f32),
    ],
)


def _k4(h4_ref, sm_ref, sq_ref, g_ref, be_ref, wo_ref, bo_ref, o_ref):
    sc, sh = _bn_scale_shift(sm_ref[...], sq_ref[...], g_ref[...], be_ref[...])
    h = h4_ref[...] * sc + sh
    o_ref[...] = jnp.dot(h, wo_ref[...], preferred_element_type=_f32) + bo_ref[...]


_k4_call = pl.pallas_call(
    _k4,
    grid=(GN,),
    in_specs=[
        pl.BlockSpec((R, DH), lambda i: (i, 0)),
        pl.BlockSpec((1, DH), _full),
        pl.BlockSpec((1, DH), _full),
        pl.BlockSpec((1, DH), _full),
        pl.BlockSpec((1, DH), _full),
        pl.BlockSpec((DH, C_OUT), _full),
        pl.BlockSpec((1, C_OUT), _full),
    ],
    out_specs=[pl.BlockSpec((R, C_OUT), lambda i: (i, 0))],
    out_shape=[jax.ShapeDtypeStruct((N, C_OUT), _f32)],
)


def kernel(x, edge_index, W_pre0, b_pre0, W_pre1, b_pre1, g_bn1, be_bn1,
           W_conv, b_conv, W_aft, b_aft, g_bn2, be_bn2, W_out, b_out):
    deg_init = jnp.concatenate(
        [jnp.ones((1, RPT, 1), _f32), jnp.zeros((1, RPT, 1), _f32)])
    ones_eg = jnp.ones((DEG_EG, 1), _f32)
    zrows = jnp.zeros((RPT, HF), _f32)

    degA, degB = _deg_call(edge_index, deg_init, ones_eg)
    h2, sm1, sq1 = _k1_call(
        x, W_pre0.T, b_pre0.reshape(1, DH), W_pre1.T, b_pre1.reshape(1, DH))
    g1 = g_bn1.reshape(1, DH)
    be1 = be_bn1.reshape(1, DH)
    (hp_tab,) = _k2_call(h2, degA, degB, sm1, sq1, g1, be1)
    sA, sB = _gat_call(edge_index, hp_tab, zrows)
    h4, sm2, sq2 = _k3_call(
        h2, sA, sB, degA, degB, sm1, sq1, g1, be1,
        W_conv.T, b_conv.reshape(1, DH), W_aft.T, b_aft.reshape(1, DH))
    (out,) = _k4_call(
        h4, sm2, sq2, g_bn2.reshape(1, DH), be_bn2.reshape(1, DH),
        W_out.T, b_out.reshape(1, C_OUT))
    return out
